# Initial kernel scaffold; baseline (speedup 1.0000x reference)
import jax
import jax.numpy as jnp
from jax.experimental import pallas as pl


def kernel(edge_index, nodes, Wg, bg, bn1_g, bn1_b, W1, b1, W2, bn2_g, bn2_b, cls_W, cls_b):
    raise NotImplementedError("write your pallas kernel here")


# speedup vs baseline: 45.6823x; 45.6823x over previous
import functools

import jax
import jax.numpy as jnp
from jax import lax
from jax.experimental import pallas as pl
from jax.experimental.pallas import tpu as pltpu
from jax.experimental.pallas import tpu_sc as plsc

N = 100000
E = 3200000
EMB = 16
MULT = 4
DEPTH = 2
NUMCLS = 40

NC = 2
NS = 16
NW = NC * NS
CH = 128
G = 16
NG = 49
C = G * NG
EPW = C * CH
EPAD = NW * EPW - E

H = N // 2
AH = 50176
NDUM = AH - H
TRH = AH // NS
LTR = H - (NS - 1) * TRH
ZB = 392

RB = 2000
NB = N // RB
HB = NB // 2

_mesh = plsc.VectorSubcoreMesh(core_axis_name="c", subcore_axis_name="s")
_sc_params = pltpu.CompilerParams(use_tc_tiling_on_sc=False)


def _zero_acc(zbuf, acc, s):
    def zb(i, carry):
        zbuf[i] = jnp.zeros((EMB,), jnp.float32)
        return carry
    lax.fori_loop(0, ZB, zb, None)
    base = s * TRH
    for k in range(TRH // ZB):
        pltpu.sync_copy(zbuf, acc.at[pl.ds(base + k * ZB, ZB)])


def _dump_acc(acc, out_hbm, c, s):
    base = s * TRH

    @pl.when(s == NS - 1)
    def _():
        pltpu.sync_copy(acc.at[pl.ds((NS - 1) * TRH, LTR)],
                        out_hbm.at[c, pl.ds((NS - 1) * TRH, LTR)])

    @pl.when(s != NS - 1)
    def _():
        pltpu.sync_copy(acc.at[pl.ds(base, TRH)],
                        out_hbm.at[c, pl.ds(base, TRH)])


@functools.partial(
    pl.kernel,
    out_type=jax.ShapeDtypeStruct((NC, H, EMB), jnp.float32),
    mesh=_mesh,
    scratch_types=[
        pltpu.VMEM((G, CH), jnp.int32),
        pltpu.VMEM((CH, EMB), jnp.float32),
        pltpu.VMEM((ZB, EMB), jnp.float32),
        pltpu.VMEM_SHARED((AH, EMB), jnp.float32),
        pltpu.SemaphoreType.DMA,
    ],
    compiler_params=_sc_params,
)
def _sc_degree(dst_hbm, out_hbm, dsti, ones, zbuf, acc, sems):
    c = lax.axis_index("c")
    s = lax.axis_index("s")
    wid = c * NS + s

    def ob(i, carry):
        ones[i] = jnp.ones((EMB,), jnp.float32)
        return carry
    lax.fori_loop(0, CH, ob, None)
    _zero_acc(zbuf, acc, s)
    plsc.subcore_barrier()

    def grp(g, carry):
        pltpu.sync_copy(dst_hbm.at[wid, pl.ds(g * G, G)], dsti)
        descs = [pltpu.async_copy(ones, acc.at[dsti.at[j]], sems, add=True)
                 for j in range(G)]
        for d in descs:
            d.wait()
        return carry
    lax.fori_loop(0, NG, grp, None)
    plsc.subcore_barrier()
    _dump_acc(acc, out_hbm, c, s)


@functools.partial(
    pl.kernel,
    out_type=jax.ShapeDtypeStruct((NC, H, EMB), jnp.float32),
    mesh=_mesh,
    scratch_types=[
        pltpu.VMEM((G, CH), jnp.int32),
        pltpu.VMEM((G, CH), jnp.int32),
        pltpu.VMEM((G, CH, EMB), jnp.float32),
        pltpu.VMEM((ZB, EMB), jnp.float32),
        pltpu.VMEM_SHARED((AH, EMB), jnp.float32),
        pltpu.SemaphoreType.DMA,
        pltpu.SemaphoreType.DMA,
    ],
    compiler_params=_sc_params,
)
def _sc_aggregate(src_hbm, dst_hbm, y_hbm, out_hbm, srci, dsti, rows, zbuf,
                  acc, semg, sems):
    c = lax.axis_index("c")
    s = lax.axis_index("s")
    wid = c * NS + s
    _zero_acc(zbuf, acc, s)
    plsc.subcore_barrier()

    def grp(g, carry):
        pltpu.sync_copy(src_hbm.at[wid, pl.ds(g * G, G)], srci)
        pltpu.sync_copy(dst_hbm.at[wid, pl.ds(g * G, G)], dsti)
        gd = [pltpu.async_copy(y_hbm.at[srci.at[j]], rows.at[j], semg)
              for j in range(G)]
        for d in gd:
            d.wait()
        sd = [pltpu.async_copy(rows.at[j], acc.at[dsti.at[j]], sems, add=True)
              for j in range(G)]
        for d in sd:
            d.wait()
        return carry
    lax.fori_loop(0, NG, grp, None)
    plsc.subcore_barrier()
    _dump_acc(acc, out_hbm, c, s)


def _stats_update(st_ref, u):
    part = jnp.concatenate(
        [jnp.sum(u, axis=0, keepdims=True),
         jnp.sum(u * u, axis=0, keepdims=True),
         jnp.zeros((6, EMB), jnp.float32)], axis=0)
    i = pl.program_id(0)

    @pl.when(i == 0)
    def _():
        st_ref[...] = part

    @pl.when(i != 0)
    def _():
        st_ref[...] = st_ref[...] + part


def _halves(i, lo0, lo1, hi0, hi1):
    return jnp.where(i < HB, lo0[0] + lo1[0], hi0[0] + hi1[0])


def _prep_body(dl0, dl1, dh0, dh1, nd, y0_ref, dd_ref):
    cnt = _halves(pl.program_id(0), dl0, dl1, dh0, dh1)[:, 0:1]
    deg = cnt + 1.0
    dinv = lax.rsqrt(deg)
    dgi = 1.0 / deg
    y0_ref[...] = nd[...] * dinv
    col = lax.broadcasted_iota(jnp.int32, (RB, EMB), 1)
    dd_ref[...] = jnp.where(col == 0, dinv, jnp.where(col == 1, dgi, 0.0))


_lo_spec = pl.BlockSpec((1, RB, EMB), lambda i: (0, i % HB, 0))
_lo_spec1 = pl.BlockSpec((1, RB, EMB), lambda i: (1, i % HB, 0))
_row_spec = pl.BlockSpec((RB, EMB), lambda i: (i, 0))
_st_spec = pl.BlockSpec((8, EMB), lambda i: (0, 0))
_half_specs = [_lo_spec, _lo_spec1, _lo_spec, _lo_spec1]


def _const_spec(shape):
    return pl.BlockSpec(shape, lambda i: tuple(0 for _ in shape))


_tc_prep = pl.pallas_call(
    _prep_body,
    grid=(NB,),
    in_specs=_half_specs + [_row_spec],
    out_specs=[_row_spec, _row_spec],
    out_shape=[jax.ShapeDtypeStruct((N, EMB), jnp.float32),
               jax.ShapeDtypeStruct((N, EMB), jnp.float32)],
)


def _mix_body(sl0, sl1, sh0, sh1, x, dd, wg, bg, u_ref, st_ref):
    sv = _halves(pl.program_id(0), sl0, sl1, sh0, sh1)
    ddv = dd[...]
    agg = sv * ddv[:, 0:1] + x[...] * ddv[:, 1:2]
    h = jnp.maximum(
        jnp.dot(agg, wg[...], preferred_element_type=jnp.float32) + bg[...],
        0.0)
    u = h + x[...]
    u_ref[...] = u
    _stats_update(st_ref, u)


_tc_mix = pl.pallas_call(
    _mix_body,
    grid=(NB,),
    in_specs=_half_specs + [_row_spec, _row_spec,
                            _const_spec((EMB, EMB)), _const_spec((1, EMB))],
    out_specs=[_row_spec, _st_spec],
    out_shape=[jax.ShapeDtypeStruct((N, EMB), jnp.float32),
               jax.ShapeDtypeStruct((8, EMB), jnp.float32)],
)


def _ff_body(u, p, w1, b1, w2, u2_ref, st_ref):
    pv = p[...]
    xp = u[...] * pv[0:1, :] + pv[1:2, :]
    h = jnp.maximum(
        jnp.dot(xp, w1[...], preferred_element_type=jnp.float32) + b1[...],
        0.0)
    u2 = jnp.dot(h, w2[...], preferred_element_type=jnp.float32) + xp
    u2_ref[...] = u2
    _stats_update(st_ref, u2)


_tc_ff = pl.pallas_call(
    _ff_body,
    grid=(NB,),
    in_specs=[_row_spec, _const_spec((8, EMB)),
              _const_spec((EMB, MULT * EMB)), _const_spec((1, MULT * EMB)),
              _const_spec((MULT * EMB, EMB))],
    out_specs=[_row_spec, _st_spec],
    out_shape=[jax.ShapeDtypeStruct((N, EMB), jnp.float32),
               jax.ShapeDtypeStruct((8, EMB), jnp.float32)],
)


def _bn_body(u2, p, dd, x_ref, y_ref):
    pv = p[...]
    xv = u2[...] * pv[0:1, :] + pv[1:2, :]
    x_ref[...] = xv
    y_ref[...] = xv * dd[...][:, 0:1]


_tc_bn = pl.pallas_call(
    _bn_body,
    grid=(NB,),
    in_specs=[_row_spec, _const_spec((8, EMB)), _row_spec],
    out_specs=[_row_spec, _row_spec],
    out_shape=[jax.ShapeDtypeStruct((N, EMB), jnp.float32),
               jax.ShapeDtypeStruct((N, EMB), jnp.float32)],
)


def _cls_body(u2, p, cw, cb, o_ref):
    pv = p[...]
    xv = u2[...] * pv[0:1, :] + pv[1:2, :]
    o_ref[...] = (jnp.dot(xv, cw[...], preferred_element_type=jnp.float32)
                  + cb[...])


_tc_cls = pl.pallas_call(
    _cls_body,
    grid=(NB,),
    in_specs=[_row_spec, _const_spec((8, EMB)),
              _const_spec((EMB, NUMCLS)), _const_spec((1, NUMCLS))],
    out_specs=pl.BlockSpec((RB, NUMCLS), lambda i: (i, 0)),
    out_shape=jax.ShapeDtypeStruct((N, NUMCLS), jnp.float32),
)


def _bn_params(st, g, b):
    m = st[0] / N
    v = st[1] / N - m * m
    a = g / jnp.sqrt(v + 1e-5)
    cc = b - m * a
    return jnp.concatenate(
        [a[None], cc[None], jnp.zeros((6, EMB), jnp.float32)], axis=0)


def kernel(edge_index, nodes, Wg, bg, bn1_g, bn1_b, W1, b1, W2, bn2_g, bn2_b,
           cls_W, cls_b):
    src = edge_index[0]
    dst = edge_index[1]
    pad = jnp.arange(EPAD, dtype=jnp.int32)
    src_p = jnp.concatenate([src, pad % N]).reshape(NW, C, CH)
    dstf = jnp.concatenate([dst, jnp.full((EPAD,), 1 << 30, jnp.int32)])
    dum = H + (jnp.arange(NW * EPW, dtype=jnp.int32) % NDUM)
    dst_lo = jnp.where(dstf < H, dstf, dum).reshape(NW, C, CH)
    dst_hi = jnp.where((dstf >= H) & (dstf < N), dstf - H, dum)
    dst_hi = dst_hi.reshape(NW, C, CH)

    deg_lo = _sc_degree(dst_lo)
    deg_hi = _sc_degree(dst_hi)
    y, dd = _tc_prep(deg_lo, deg_lo, deg_hi, deg_hi, nodes)

    x = nodes
    for i in range(DEPTH):
        s_lo = _sc_aggregate(src_p, dst_lo, y)
        s_hi = _sc_aggregate(src_p, dst_hi, y)
        u1, st1 = _tc_mix(s_lo, s_lo, s_hi, s_hi, x, dd, Wg[i], bg[i][None])
        p1 = _bn_params(st1, bn1_g[i], bn1_b[i])
        u2, st2 = _tc_ff(u1, p1, W1[i], b1[i][None], W2[i])
        p2 = _bn_params(st2, bn2_g[i], bn2_b[i])
        if i < DEPTH - 1:
            x, y = _tc_bn(u2, p2, dd)
        else:
            return _tc_cls(u2, p2, cls_W, cls_b[None])


# speedup vs baseline: 57.9064x; 1.2676x over previous
import functools

import jax
import jax.numpy as jnp
from jax import lax
from jax.experimental import pallas as pl
from jax.experimental.pallas import tpu as pltpu
from jax.experimental.pallas import tpu_sc as plsc

N = 100000
E = 3200000
EMB = 16
MULT = 4
DEPTH = 2
NUMCLS = 40

NC = 2
NS = 16
NW = NC * NS
CH = 128
G = 16
NG = 49
C = G * NG
EPW = C * CH
EPAD = NW * EPW - E

H = N // 2
AH = 50176
NDUM = AH - H
TRH = AH // NS
LTR = H - (NS - 1) * TRH
ZB = 392

RB = 2000
NB = N // RB
HB = NB // 2

_mesh = plsc.VectorSubcoreMesh(core_axis_name="c", subcore_axis_name="s")
_sc_params = pltpu.CompilerParams(use_tc_tiling_on_sc=False)


def _zero_acc(zbuf, acc, s):
    def zb(i, carry):
        zbuf[i] = jnp.zeros((EMB,), jnp.float32)
        return carry
    lax.fori_loop(0, ZB, zb, None)
    base = s * TRH
    for k in range(TRH // ZB):
        pltpu.sync_copy(zbuf, acc.at[pl.ds(base + k * ZB, ZB)])


def _dump_acc(acc, out_hbm, c, s):
    base = s * TRH

    @pl.when(s == NS - 1)
    def _():
        pltpu.sync_copy(acc.at[pl.ds((NS - 1) * TRH, LTR)],
                        out_hbm.at[c, pl.ds((NS - 1) * TRH, LTR)])

    @pl.when(s != NS - 1)
    def _():
        pltpu.sync_copy(acc.at[pl.ds(base, TRH)],
                        out_hbm.at[c, pl.ds(base, TRH)])


@functools.partial(
    pl.kernel,
    out_type=jax.ShapeDtypeStruct((NC, H, EMB), jnp.float32),
    mesh=_mesh,
    scratch_types=[
        pltpu.VMEM((G, CH), jnp.int32),
        pltpu.VMEM((G, CH), jnp.int32),
        pltpu.VMEM((CH, EMB), jnp.float32),
        pltpu.VMEM((ZB, EMB), jnp.float32),
        pltpu.VMEM_SHARED((AH, EMB), jnp.float32),
        pltpu.SemaphoreType.DMA,
    ],
    compiler_params=_sc_params,
)
def _sc_degree(dst_hbm, out_hbm, dsti0, dsti1, ones, zbuf, acc, sems):
    c = lax.axis_index("c")
    s = lax.axis_index("s")
    wid = c * NS + s
    dsti = [dsti0, dsti1]

    def ob(i, carry):
        ones[i] = jnp.ones((EMB,), jnp.float32)
        return carry
    lax.fori_loop(0, CH, ob, None)
    _zero_acc(zbuf, acc, s)
    plsc.subcore_barrier()

    def fire(g, b):
        for j in range(G):
            pltpu.async_copy(ones, acc.at[dsti[b].at[j]], sems, add=True)

    def drain(b):
        for j in range(G):
            pltpu.make_async_copy(ones, acc.at[dsti[b].at[j]], sems).wait()

    pltpu.sync_copy(dst_hbm.at[wid, pl.ds(0, G)], dsti0)

    def grp(k, carry):
        g = 2 * k
        fire(g, 0)
        pltpu.sync_copy(dst_hbm.at[wid, pl.ds((g + 1) * G, G)], dsti1)
        drain(0)
        fire(g + 1, 1)
        pltpu.sync_copy(dst_hbm.at[wid, pl.ds((g + 2) * G, G)], dsti0)
        drain(1)
        return carry
    lax.fori_loop(0, (NG - 1) // 2, grp, None)
    fire(NG - 1, 0)
    drain(0)
    plsc.subcore_barrier()
    _dump_acc(acc, out_hbm, c, s)


@functools.partial(
    pl.kernel,
    out_type=jax.ShapeDtypeStruct((NC, H, EMB), jnp.float32),
    mesh=_mesh,
    scratch_types=[
        pltpu.VMEM((G, CH), jnp.int32),
        pltpu.VMEM((G, CH), jnp.int32),
        pltpu.VMEM((G, CH), jnp.int32),
        pltpu.VMEM((G, CH), jnp.int32),
        pltpu.VMEM((G, CH, EMB), jnp.float32),
        pltpu.VMEM((G, CH, EMB), jnp.float32),
        pltpu.VMEM((ZB, EMB), jnp.float32),
        pltpu.VMEM_SHARED((AH, EMB), jnp.float32),
        pltpu.SemaphoreType.DMA,
        pltpu.SemaphoreType.DMA,
    ],
    compiler_params=_sc_params,
)
def _sc_aggregate(src_hbm, dst_hbm, y_hbm, out_hbm, srci0, srci1, dsti0,
                  dsti1, rows0, rows1, zbuf, acc, semg, sems):
    c = lax.axis_index("c")
    s = lax.axis_index("s")
    wid = c * NS + s
    srci = [srci0, srci1]
    dsti = [dsti0, dsti1]
    rows = [rows0, rows1]
    _zero_acc(zbuf, acc, s)
    plsc.subcore_barrier()

    def fire_gather(g, b):
        pltpu.sync_copy(src_hbm.at[wid, pl.ds(g * G, G)], srci[b])
        pltpu.sync_copy(dst_hbm.at[wid, pl.ds(g * G, G)], dsti[b])
        for j in range(G):
            pltpu.async_copy(y_hbm.at[srci[b].at[j]], rows[b].at[j], semg)

    def drain_gather(b):
        for j in range(G):
            pltpu.make_async_copy(y_hbm.at[srci[b].at[j]], rows[b].at[j],
                                  semg).wait()

    def fire_scatter(b):
        for j in range(G):
            pltpu.async_copy(rows[b].at[j], acc.at[dsti[b].at[j]], sems,
                             add=True)

    def drain_scatter(b):
        for j in range(G):
            pltpu.make_async_copy(rows[b].at[j], acc.at[dsti[b].at[j]],
                                  sems).wait()

    fire_gather(0, 0)

    def grp(k, carry):
        g = 2 * k
        fire_gather(g + 1, 1)
        drain_gather(0)
        fire_scatter(0)
        drain_scatter(0)
        fire_gather(g + 2, 0)
        drain_gather(1)
        fire_scatter(1)
        drain_scatter(1)
        return carry
    lax.fori_loop(0, (NG - 1) // 2, grp, None)
    drain_gather(0)
    fire_scatter(0)
    drain_scatter(0)
    plsc.subcore_barrier()
    _dump_acc(acc, out_hbm, c, s)


def _stats_update(st_ref, u):
    part = jnp.concatenate(
        [jnp.sum(u, axis=0, keepdims=True),
         jnp.sum(u * u, axis=0, keepdims=True),
         jnp.zeros((6, EMB), jnp.float32)], axis=0)
    i = pl.program_id(0)

    @pl.when(i == 0)
    def _():
        st_ref[...] = part

    @pl.when(i != 0)
    def _():
        st_ref[...] = st_ref[...] + part


def _halves(i, lo0, lo1, hi0, hi1):
    return jnp.where(i < HB, lo0[0] + lo1[0], hi0[0] + hi1[0])


def _prep_body(dl0, dl1, dh0, dh1, nd, y0_ref, dd_ref):
    cnt = _halves(pl.program_id(0), dl0, dl1, dh0, dh1)[:, 0:1]
    deg = cnt + 1.0
    dinv = lax.rsqrt(deg)
    dgi = 1.0 / deg
    y0_ref[...] = nd[...] * dinv
    col = lax.broadcasted_iota(jnp.int32, (RB, EMB), 1)
    dd_ref[...] = jnp.where(col == 0, dinv, jnp.where(col == 1, dgi, 0.0))


_lo_spec = pl.BlockSpec((1, RB, EMB), lambda i: (0, i % HB, 0))
_lo_spec1 = pl.BlockSpec((1, RB, EMB), lambda i: (1, i % HB, 0))
_row_spec = pl.BlockSpec((RB, EMB), lambda i: (i, 0))
_st_spec = pl.BlockSpec((8, EMB), lambda i: (0, 0))
_half_specs = [_lo_spec, _lo_spec1, _lo_spec, _lo_spec1]


def _const_spec(shape):
    return pl.BlockSpec(shape, lambda i: tuple(0 for _ in shape))


_tc_prep = pl.pallas_call(
    _prep_body,
    grid=(NB,),
    in_specs=_half_specs + [_row_spec],
    out_specs=[_row_spec, _row_spec],
    out_shape=[jax.ShapeDtypeStruct((N, EMB), jnp.float32),
               jax.ShapeDtypeStruct((N, EMB), jnp.float32)],
)


def _mix_body(sl0, sl1, sh0, sh1, x, dd, wg, bg, u_ref, st_ref):
    sv = _halves(pl.program_id(0), sl0, sl1, sh0, sh1)
    ddv = dd[...]
    agg = sv * ddv[:, 0:1] + x[...] * ddv[:, 1:2]
    h = jnp.maximum(
        jnp.dot(agg, wg[...], preferred_element_type=jnp.float32) + bg[...],
        0.0)
    u = h + x[...]
    u_ref[...] = u
    _stats_update(st_ref, u)


_tc_mix = pl.pallas_call(
    _mix_body,
    grid=(NB,),
    in_specs=_half_specs + [_row_spec, _row_spec,
                            _const_spec((EMB, EMB)), _const_spec((1, EMB))],
    out_specs=[_row_spec, _st_spec],
    out_shape=[jax.ShapeDtypeStruct((N, EMB), jnp.float32),
               jax.ShapeDtypeStruct((8, EMB), jnp.float32)],
)


def _ff_body(u, p, w1, b1, w2, u2_ref, st_ref):
    pv = p[...]
    xp = u[...] * pv[0:1, :] + pv[1:2, :]
    h = jnp.maximum(
        jnp.dot(xp, w1[...], preferred_element_type=jnp.float32) + b1[...],
        0.0)
    u2 = jnp.dot(h, w2[...], preferred_element_type=jnp.float32) + xp
    u2_ref[...] = u2
    _stats_update(st_ref, u2)


_tc_ff = pl.pallas_call(
    _ff_body,
    grid=(NB,),
    in_specs=[_row_spec, _const_spec((8, EMB)),
              _const_spec((EMB, MULT * EMB)), _const_spec((1, MULT * EMB)),
              _const_spec((MULT * EMB, EMB))],
    out_specs=[_row_spec, _st_spec],
    out_shape=[jax.ShapeDtypeStruct((N, EMB), jnp.float32),
               jax.ShapeDtypeStruct((8, EMB), jnp.float32)],
)


def _bn_body(u2, p, dd, x_ref, y_ref):
    pv = p[...]
    xv = u2[...] * pv[0:1, :] + pv[1:2, :]
    x_ref[...] = xv
    y_ref[...] = xv * dd[...][:, 0:1]


_tc_bn = pl.pallas_call(
    _bn_body,
    grid=(NB,),
    in_specs=[_row_spec, _const_spec((8, EMB)), _row_spec],
    out_specs=[_row_spec, _row_spec],
    out_shape=[jax.ShapeDtypeStruct((N, EMB), jnp.float32),
               jax.ShapeDtypeStruct((N, EMB), jnp.float32)],
)


def _cls_body(u2, p, cw, cb, o_ref):
    pv = p[...]
    xv = u2[...] * pv[0:1, :] + pv[1:2, :]
    o_ref[...] = (jnp.dot(xv, cw[...], preferred_element_type=jnp.float32)
                  + cb[...])


_tc_cls = pl.pallas_call(
    _cls_body,
    grid=(NB,),
    in_specs=[_row_spec, _const_spec((8, EMB)),
              _const_spec((EMB, NUMCLS)), _const_spec((1, NUMCLS))],
    out_specs=pl.BlockSpec((RB, NUMCLS), lambda i: (i, 0)),
    out_shape=jax.ShapeDtypeStruct((N, NUMCLS), jnp.float32),
)


def _bn_params(st, g, b):
    m = st[0] / N
    v = st[1] / N - m * m
    a = g / jnp.sqrt(v + 1e-5)
    cc = b - m * a
    return jnp.concatenate(
        [a[None], cc[None], jnp.zeros((6, EMB), jnp.float32)], axis=0)


def kernel(edge_index, nodes, Wg, bg, bn1_g, bn1_b, W1, b1, W2, bn2_g, bn2_b,
           cls_W, cls_b):
    src = edge_index[0]
    dst = edge_index[1]
    pad = jnp.arange(EPAD, dtype=jnp.int32)
    src_p = jnp.concatenate([src, pad % N]).reshape(NW, C, CH)
    dstf = jnp.concatenate([dst, jnp.full((EPAD,), 1 << 30, jnp.int32)])
    dum = H + (jnp.arange(NW * EPW, dtype=jnp.int32) % NDUM)
    dst_lo = jnp.where(dstf < H, dstf, dum).reshape(NW, C, CH)
    dst_hi = jnp.where((dstf >= H) & (dstf < N), dstf - H, dum)
    dst_hi = dst_hi.reshape(NW, C, CH)

    deg_lo = _sc_degree(dst_lo)
    deg_hi = _sc_degree(dst_hi)
    y, dd = _tc_prep(deg_lo, deg_lo, deg_hi, deg_hi, nodes)

    x = nodes
    for i in range(DEPTH):
        s_lo = _sc_aggregate(src_p, dst_lo, y)
        s_hi = _sc_aggregate(src_p, dst_hi, y)
        u1, st1 = _tc_mix(s_lo, s_lo, s_hi, s_hi, x, dd, Wg[i], bg[i][None])
        p1 = _bn_params(st1, bn1_g[i], bn1_b[i])
        u2, st2 = _tc_ff(u1, p1, W1[i], b1[i][None], W2[i])
        p2 = _bn_params(st2, bn2_g[i], bn2_b[i])
        if i < DEPTH - 1:
            x, y = _tc_bn(u2, p2, dd)
        else:
            return _tc_cls(u2, p2, cls_W, cls_b[None])


# speedup vs baseline: 62.6287x; 1.0815x over previous
import functools

import jax
import jax.numpy as jnp
from jax import lax
from jax.experimental import pallas as pl
from jax.experimental.pallas import tpu as pltpu
from jax.experimental.pallas import tpu_sc as plsc

N = 100000
E = 3200000
EMB = 16
MULT = 4
DEPTH = 2
NUMCLS = 40

NC = 2
NS = 16
NW = NC * NS
CH = 128
G = 16
NG = 49
C = G * NG
EPW = C * CH
EPAD = NW * EPW - E

H = N // 2
AH = 50176
NDUM = AH - H
TRH = AH // NS
LTR = H - (NS - 1) * TRH
ZB = 392

RB = 5000
NB = N // RB
HB = NB // 2

_mesh = plsc.VectorSubcoreMesh(core_axis_name="c", subcore_axis_name="s")
_sc_params = pltpu.CompilerParams(use_tc_tiling_on_sc=False)


def _zero_acc(zbuf, acc, s):
    def zb(i, carry):
        zbuf[i] = jnp.zeros((EMB,), jnp.float32)
        return carry
    lax.fori_loop(0, ZB, zb, None)
    base = s * TRH
    for k in range(TRH // ZB):
        pltpu.sync_copy(zbuf, acc.at[pl.ds(base + k * ZB, ZB)])


def _dump_acc(acc, out_hbm, c, s):
    base = s * TRH

    @pl.when(s == NS - 1)
    def _():
        pltpu.sync_copy(acc.at[pl.ds((NS - 1) * TRH, LTR)],
                        out_hbm.at[c, pl.ds((NS - 1) * TRH, LTR)])

    @pl.when(s != NS - 1)
    def _():
        pltpu.sync_copy(acc.at[pl.ds(base, TRH)],
                        out_hbm.at[c, pl.ds(base, TRH)])


@functools.partial(
    pl.kernel,
    out_type=jax.ShapeDtypeStruct((NC, H, EMB), jnp.float32),
    mesh=_mesh,
    scratch_types=[
        pltpu.VMEM((G, CH), jnp.int32),
        pltpu.VMEM((G, CH), jnp.int32),
        pltpu.VMEM((CH, EMB), jnp.float32),
        pltpu.VMEM((ZB, EMB), jnp.float32),
        pltpu.VMEM_SHARED((AH, EMB), jnp.float32),
        pltpu.SemaphoreType.DMA,
    ],
    compiler_params=_sc_params,
)
def _sc_degree(dst_hbm, out_hbm, dsti0, dsti1, ones, zbuf, acc, sems):
    c = lax.axis_index("c")
    s = lax.axis_index("s")
    wid = c * NS + s
    dsti = [dsti0, dsti1]

    def ob(i, carry):
        ones[i] = jnp.ones((EMB,), jnp.float32)
        return carry
    lax.fori_loop(0, CH, ob, None)
    _zero_acc(zbuf, acc, s)
    plsc.subcore_barrier()

    def fire(g, b):
        for j in range(G):
            pltpu.async_copy(ones, acc.at[dsti[b].at[j]], sems, add=True)

    def drain(b):
        for j in range(G):
            pltpu.make_async_copy(ones, acc.at[dsti[b].at[j]], sems).wait()

    pltpu.sync_copy(dst_hbm.at[wid, pl.ds(0, G)], dsti0)

    def grp(k, carry):
        g = 2 * k
        fire(g, 0)
        pltpu.sync_copy(dst_hbm.at[wid, pl.ds((g + 1) * G, G)], dsti1)
        drain(0)
        fire(g + 1, 1)
        pltpu.sync_copy(dst_hbm.at[wid, pl.ds((g + 2) * G, G)], dsti0)
        drain(1)
        return carry
    lax.fori_loop(0, (NG - 1) // 2, grp, None)
    fire(NG - 1, 0)
    drain(0)
    plsc.subcore_barrier()
    _dump_acc(acc, out_hbm, c, s)


@functools.partial(
    pl.kernel,
    out_type=jax.ShapeDtypeStruct((NC, H, EMB), jnp.float32),
    mesh=_mesh,
    scratch_types=[
        pltpu.VMEM((G, CH), jnp.int32),
        pltpu.VMEM((G, CH), jnp.int32),
        pltpu.VMEM((G, CH), jnp.int32),
        pltpu.VMEM((G, CH), jnp.int32),
        pltpu.VMEM((G, CH, EMB), jnp.float32),
        pltpu.VMEM((G, CH, EMB), jnp.float32),
        pltpu.VMEM((ZB, EMB), jnp.float32),
        pltpu.VMEM_SHARED((AH, EMB), jnp.float32),
        pltpu.SemaphoreType.DMA,
        pltpu.SemaphoreType.DMA,
    ],
    compiler_params=_sc_params,
)
def _sc_aggregate(src_hbm, dst_hbm, y_hbm, out_hbm, srci0, srci1, dsti0,
                  dsti1, rows0, rows1, zbuf, acc, semg, sems):
    c = lax.axis_index("c")
    s = lax.axis_index("s")
    wid = c * NS + s
    srci = [srci0, srci1]
    dsti = [dsti0, dsti1]
    rows = [rows0, rows1]
    _zero_acc(zbuf, acc, s)
    plsc.subcore_barrier()

    def fire_gather(g, b):
        pltpu.sync_copy(src_hbm.at[wid, pl.ds(g * G, G)], srci[b])
        pltpu.sync_copy(dst_hbm.at[wid, pl.ds(g * G, G)], dsti[b])
        for j in range(G):
            pltpu.async_copy(y_hbm.at[srci[b].at[j]], rows[b].at[j], semg)

    def drain_gather(b):
        for j in range(G):
            pltpu.make_async_copy(y_hbm.at[srci[b].at[j]], rows[b].at[j],
                                  semg).wait()

    def fire_scatter(b):
        for j in range(G):
            pltpu.async_copy(rows[b].at[j], acc.at[dsti[b].at[j]], sems,
                             add=True)

    def drain_scatter(b):
        for j in range(G):
            pltpu.make_async_copy(rows[b].at[j], acc.at[dsti[b].at[j]],
                                  sems).wait()

    fire_gather(0, 0)

    def grp(k, carry):
        g = 2 * k
        fire_gather(g + 1, 1)
        drain_gather(0)
        fire_scatter(0)
        drain_scatter(0)
        fire_gather(g + 2, 0)
        drain_gather(1)
        fire_scatter(1)
        drain_scatter(1)
        return carry
    lax.fori_loop(0, (NG - 1) // 2, grp, None)
    drain_gather(0)
    fire_scatter(0)
    drain_scatter(0)
    plsc.subcore_barrier()
    _dump_acc(acc, out_hbm, c, s)


def _stats_update(st_ref, u, first):
    part = jnp.concatenate(
        [jnp.sum(u, axis=0, keepdims=True),
         jnp.sum(u * u, axis=0, keepdims=True),
         jnp.zeros((6, EMB), jnp.float32)], axis=0)

    @pl.when(first)
    def _():
        st_ref[...] = part

    @pl.when(jnp.logical_not(first))
    def _():
        st_ref[...] = st_ref[...] + part


def _halves(h, lo0, lo1, hi0, hi1):
    return jnp.where(h == 0, lo0[0] + lo1[0], hi0[0] + hi1[0])


def _prep_body(dl0, dl1, dh0, dh1, nd, y0_ref, dd_ref):
    h = pl.program_id(0)
    cnt = _halves(h, dl0, dl1, dh0, dh1)[:, 0:1]
    deg = cnt + 1.0
    dinv = lax.rsqrt(deg)
    dgi = 1.0 / deg
    y0_ref[...] = nd[...] * dinv
    col = lax.broadcasted_iota(jnp.int32, (RB, EMB), 1)
    dd_ref[...] = jnp.where(col == 0, dinv, jnp.where(col == 1, dgi, 0.0))


_lo_spec = pl.BlockSpec(
    (1, RB, EMB), lambda h, i: (0, jnp.where(h == 0, i, HB - 1), 0))
_lo_spec1 = pl.BlockSpec(
    (1, RB, EMB), lambda h, i: (1, jnp.where(h == 0, i, HB - 1), 0))
_hi_spec = pl.BlockSpec(
    (1, RB, EMB), lambda h, i: (0, jnp.where(h == 0, 0, i), 0))
_hi_spec1 = pl.BlockSpec(
    (1, RB, EMB), lambda h, i: (1, jnp.where(h == 0, 0, i), 0))
_half_specs = [_lo_spec, _lo_spec1, _hi_spec, _hi_spec1]
_row2_spec = pl.BlockSpec((RB, EMB), lambda h, i: (h * HB + i, 0))
_st2_spec = pl.BlockSpec((8, EMB), lambda h, i: (0, 0))

_row_spec = pl.BlockSpec((RB, EMB), lambda i: (i, 0))
_st_spec = pl.BlockSpec((8, EMB), lambda i: (0, 0))


def _const_spec(shape, ndim=1):
    return pl.BlockSpec(shape, lambda *_: tuple(0 for _ in shape))


_tc_prep = pl.pallas_call(
    _prep_body,
    grid=(2, HB),
    in_specs=_half_specs + [_row2_spec],
    out_specs=[_row2_spec, _row2_spec],
    out_shape=[jax.ShapeDtypeStruct((N, EMB), jnp.float32),
               jax.ShapeDtypeStruct((N, EMB), jnp.float32)],
)


def _mix_body(sl0, sl1, sh0, sh1, x, dd, wg, bg, u_ref, st_ref):
    h = pl.program_id(0)
    i = pl.program_id(1)
    sv = _halves(h, sl0, sl1, sh0, sh1)
    ddv = dd[...]
    agg = sv * ddv[:, 0:1] + x[...] * ddv[:, 1:2]
    hh = jnp.maximum(
        jnp.dot(agg, wg[...], preferred_element_type=jnp.float32) + bg[...],
        0.0)
    u = hh + x[...]
    u_ref[...] = u
    _stats_update(st_ref, u, (h == 0) & (i == 0))


_tc_mix = pl.pallas_call(
    _mix_body,
    grid=(2, HB),
    in_specs=_half_specs + [_row2_spec, _row2_spec,
                            _const_spec((EMB, EMB)), _const_spec((1, EMB))],
    out_specs=[_row2_spec, _st2_spec],
    out_shape=[jax.ShapeDtypeStruct((N, EMB), jnp.float32),
               jax.ShapeDtypeStruct((8, EMB), jnp.float32)],
)


def _ff_body(u, p, w1, b1, w2, u2_ref, st_ref):
    pv = p[...]
    xp = u[...] * pv[0:1, :] + pv[1:2, :]
    h = jnp.maximum(
        jnp.dot(xp, w1[...], preferred_element_type=jnp.float32) + b1[...],
        0.0)
    u2 = jnp.dot(h, w2[...], preferred_element_type=jnp.float32) + xp
    u2_ref[...] = u2
    _stats_update(st_ref, u2, pl.program_id(0) == 0)


_tc_ff = pl.pallas_call(
    _ff_body,
    grid=(NB,),
    in_specs=[_row_spec, _const_spec((8, EMB)),
              _const_spec((EMB, MULT * EMB)), _const_spec((1, MULT * EMB)),
              _const_spec((MULT * EMB, EMB))],
    out_specs=[_row_spec, _st_spec],
    out_shape=[jax.ShapeDtypeStruct((N, EMB), jnp.float32),
               jax.ShapeDtypeStruct((8, EMB), jnp.float32)],
)


def _bn_body(u2, p, dd, x_ref, y_ref):
    pv = p[...]
    xv = u2[...] * pv[0:1, :] + pv[1:2, :]
    x_ref[...] = xv
    y_ref[...] = xv * dd[...][:, 0:1]


_tc_bn = pl.pallas_call(
    _bn_body,
    grid=(NB,),
    in_specs=[_row_spec, _const_spec((8, EMB)), _row_spec],
    out_specs=[_row_spec, _row_spec],
    out_shape=[jax.ShapeDtypeStruct((N, EMB), jnp.float32),
               jax.ShapeDtypeStruct((N, EMB), jnp.float32)],
)


def _cls_body(u2, p, cw, cb, o_ref):
    pv = p[...]
    xv = u2[...] * pv[0:1, :] + pv[1:2, :]
    o_ref[...] = (jnp.dot(xv, cw[...], preferred_element_type=jnp.float32)
                  + cb[...])


_tc_cls = pl.pallas_call(
    _cls_body,
    grid=(NB,),
    in_specs=[_row_spec, _const_spec((8, EMB)),
              _const_spec((EMB, NUMCLS)), _const_spec((1, NUMCLS))],
    out_specs=pl.BlockSpec((RB, NUMCLS), lambda i: (i, 0)),
    out_shape=jax.ShapeDtypeStruct((N, NUMCLS), jnp.float32),
)


def _bn_params(st, g, b):
    m = st[0] / N
    v = st[1] / N - m * m
    a = g / jnp.sqrt(v + 1e-5)
    cc = b - m * a
    return jnp.concatenate(
        [a[None], cc[None], jnp.zeros((6, EMB), jnp.float32)], axis=0)


def kernel(edge_index, nodes, Wg, bg, bn1_g, bn1_b, W1, b1, W2, bn2_g, bn2_b,
           cls_W, cls_b):
    src = edge_index[0]
    dst = edge_index[1]
    pad = jnp.arange(EPAD, dtype=jnp.int32)
    src_p = jnp.concatenate([src, pad % N]).reshape(NW, C, CH)
    dstf = jnp.concatenate([dst, jnp.full((EPAD,), 1 << 30, jnp.int32)])
    dum = H + (jnp.arange(NW * EPW, dtype=jnp.int32) % NDUM)
    dst_lo = jnp.where(dstf < H, dstf, dum).reshape(NW, C, CH)
    dst_hi = jnp.where((dstf >= H) & (dstf < N), dstf - H, dum)
    dst_hi = dst_hi.reshape(NW, C, CH)

    deg_lo = _sc_degree(dst_lo)
    deg_hi = _sc_degree(dst_hi)
    y, dd = _tc_prep(deg_lo, deg_lo, deg_hi, deg_hi, nodes)

    x = nodes
    for i in range(DEPTH):
        s_lo = _sc_aggregate(src_p, dst_lo, y)
        s_hi = _sc_aggregate(src_p, dst_hi, y)
        u1, st1 = _tc_mix(s_lo, s_lo, s_hi, s_hi, x, dd, Wg[i], bg[i][None])
        p1 = _bn_params(st1, bn1_g[i], bn1_b[i])
        u2, st2 = _tc_ff(u1, p1, W1[i], b1[i][None], W2[i])
        p2 = _bn_params(st2, bn2_g[i], bn2_b[i])
        if i < DEPTH - 1:
            x, y = _tc_bn(u2, p2, dd)
        else:
            return _tc_cls(u2, p2, cls_W, cls_b[None])
